# baseline (speedup 1.0000x reference)
import functools

import jax
import jax.numpy as jnp
from jax import lax
from jax.experimental import pallas as pl
from jax.experimental.pallas import tpu as pltpu
from jax.experimental.pallas import tpu_sc as plsc

VOCAB = 30
D = 1024
B = 32 * 1024

NC = 2
NS = 16
NW = NC * NS
B_PER_W = B // NW
LANES = 16
CHUNK = 32
NCHUNK = B_PER_W // CHUNK
NBUF = 2

_mesh = plsc.VectorSubcoreMesh(
    core_axis_name="c", subcore_axis_name="s", num_cores=NC, num_subcores=NS
)


@functools.partial(
    pl.kernel,
    out_type=jax.ShapeDtypeStruct((B, D), jnp.float32),
    mesh=_mesh,
    scratch_types=[
        pltpu.VMEM((B_PER_W,), jnp.int32),
        pltpu.VMEM((VOCAB, D), jnp.float32),
        pltpu.VMEM((NBUF * CHUNK, D), jnp.float32),
        pltpu.SemaphoreType.DMA,
    ],
)
def _embed(x_hbm, table_hbm, out_hbm, idx_v, table_v, stage_v, wsem):
    wid = lax.axis_index("s") * NC + lax.axis_index("c")
    base = wid * B_PER_W
    pltpu.sync_copy(x_hbm.at[wid], idx_v)
    pltpu.sync_copy(table_hbm, table_v)

    def fill(c, b):
        rows = []
        for h in range(CHUNK // LANES):
            xv = idx_v[pl.ds(c * CHUNK + h * LANES, LANES)]
            rows.extend(xv[j] for j in range(LANES))

        def cp(k, carry):
            for j in range(CHUNK):
                stage_v[b * CHUNK + j, pl.ds(k * LANES, LANES)] = (
                    table_v[rows[j], pl.ds(k * LANES, LANES)])
            return carry

        lax.fori_loop(0, D // LANES, cp, 0)

    def big_start(c, b):
        pltpu.async_copy(
            stage_v.at[pl.ds(b * CHUNK, CHUNK)],
            out_hbm.at[pl.ds(base + c * CHUNK, CHUNK)], wsem)

    def big_wait():
        pltpu.make_async_copy(
            stage_v.at[pl.ds(0, CHUNK)], out_hbm.at[pl.ds(base, CHUNK)],
            wsem).wait()

    fill(0, 0)
    big_start(0, 0)
    fill(1, 1)
    big_start(1, 1)

    def body(g, carry):
        for b in range(NBUF):
            c = 2 + g * NBUF + b
            big_wait()
            fill(c, b)
            big_start(c, b)
        return carry

    lax.fori_loop(0, (NCHUNK - 2) // NBUF, body, 0)
    big_wait()
    big_wait()


@jax.jit
def kernel(x, table):
    x_r = x.reshape(NW, B_PER_W)
    out = _embed(x_r, table)
    return out.reshape(32, 1024, D)


# speedup vs baseline: 3.6924x; 3.6924x over previous
import functools

import jax
import jax.numpy as jnp
from jax import lax
from jax.experimental import pallas as pl
from jax.experimental.pallas import tpu as pltpu
from jax.experimental.pallas import tpu_sc as plsc

VOCAB = 30
D = 1024
B = 32 * 1024

NC = 2
NS = 16
NW = NC * NS
B_PER_W = B // NW
LANES = 16
GSIZE = 16
NGROUP = B_PER_W // GSIZE

_mesh = plsc.VectorSubcoreMesh(
    core_axis_name="c", subcore_axis_name="s", num_cores=NC, num_subcores=NS
)


@functools.partial(
    pl.kernel,
    out_type=jax.ShapeDtypeStruct((B, D), jnp.float32),
    mesh=_mesh,
    scratch_types=[
        pltpu.VMEM((B_PER_W,), jnp.int32),
        pltpu.VMEM((VOCAB, D), jnp.float32),
        pltpu.SemaphoreType.DMA,
    ],
)
def _embed(x_hbm, table_hbm, out_hbm, idx_v, table_v, wsem):
    wid = lax.axis_index("s") * NC + lax.axis_index("c")
    base = wid * B_PER_W
    pltpu.sync_copy(x_hbm.at[wid], idx_v)
    pltpu.sync_copy(table_hbm, table_v)

    def issue_group(g):
        for h in range(GSIZE // LANES):
            xv = idx_v[pl.ds(g * GSIZE + h * LANES, LANES)]
            for j in range(LANES):
                i = g * GSIZE + h * LANES + j
                pltpu.async_copy(
                    table_v.at[pl.ds(xv[j], 1)],
                    out_hbm.at[pl.ds(base + i, 1)], wsem)

    def drain(n):
        pltpu.make_async_copy(
            table_v.at[pl.ds(0, n)], out_hbm.at[pl.ds(base, n)], wsem).wait()

    issue_group(0)

    def body(g, carry):
        issue_group(g + 1)
        drain(GSIZE)
        return carry

    lax.fori_loop(0, NGROUP - 1, body, 0)
    drain(GSIZE)


@jax.jit
def kernel(x, table):
    x_r = x.reshape(NW, B_PER_W)
    out = _embed(x_r, table)
    return out.reshape(32, 1024, D)
